# baseline (speedup 1.0000x reference)
import functools

import jax
import jax.numpy as jnp
from jax import lax
from jax.experimental import pallas as pl
from jax.experimental.pallas import tpu as pltpu
from jax.experimental.pallas import tpu_sc as plsc

KK = 2048


def _scores_body(h_ref, w_ref, out_ref):
    hb = h_ref[...].astype(jnp.bfloat16).astype(jnp.float32)
    wb = w_ref[...].astype(jnp.bfloat16).astype(jnp.float32)
    logits = jnp.sum(hb * wb, axis=1, keepdims=True)
    out_ref[...] = 1.0 / (1.0 + jnp.exp(-logits))


def _scores(h, w_row, interpret=False):
    n = h.shape[0]
    return pl.pallas_call(
        _scores_body,
        out_shape=jax.ShapeDtypeStruct((n, 1), jnp.float32),
        interpret=interpret,
    )(h, w_row)


def _rank_body(sc_ref, sr_ref, out_ref):
    blk, n = sc_ref.shape[0], sr_ref.shape[1]
    i0 = pl.program_id(0) * blk
    s_i = sc_ref[...]
    s_j = sr_ref[...]
    row_i = lax.broadcasted_iota(jnp.int32, (blk, n), 0) + i0
    col_j = lax.broadcasted_iota(jnp.int32, (blk, n), 1)
    gt = (s_j > s_i).astype(jnp.int32)
    tie = jnp.logical_and(s_j == s_i, col_j < row_i).astype(jnp.int32)
    out_ref[...] = jnp.sum(gt + tie, axis=1, keepdims=True)


def _rank(scores_col, scores_row, blk=512, interpret=False):
    n = scores_col.shape[0]
    return pl.pallas_call(
        _rank_body,
        grid=(n // blk,),
        in_specs=[
            pl.BlockSpec((blk, 1), lambda i: (i, 0)),
            pl.BlockSpec((1, n), lambda i: (0, 0)),
        ],
        out_specs=pl.BlockSpec((blk, 1), lambda i: (i, 0)),
        out_shape=jax.ShapeDtypeStruct((n, 1), jnp.int32),
        interpret=interpret,
    )(scores_col, scores_row)


def _select_body(rank_ref, score_ref, idx_ref, val_ref):
    blk, n = idx_ref.shape[0], rank_ref.shape[1]
    r0 = pl.program_id(0) * blk
    rk = rank_ref[...]
    sc = score_ref[...]
    r = lax.broadcasted_iota(jnp.int32, (blk, n), 0) + r0
    j = lax.broadcasted_iota(jnp.int32, (blk, n), 1)
    eq = rk == r
    idx_ref[...] = jnp.sum(jnp.where(eq, j, 0), axis=1, keepdims=True)
    val_ref[...] = jnp.sum(jnp.where(eq, sc, 0.0), axis=1, keepdims=True)


def _select(rank_row, scores_row, blk=512, interpret=False):
    n = rank_row.shape[1]
    return pl.pallas_call(
        _select_body,
        grid=(KK // blk,),
        in_specs=[
            pl.BlockSpec((1, n), lambda i: (0, 0)),
            pl.BlockSpec((1, n), lambda i: (0, 0)),
        ],
        out_specs=[
            pl.BlockSpec((blk, 1), lambda i: (i, 0)),
            pl.BlockSpec((blk, 1), lambda i: (i, 0)),
        ],
        out_shape=[
            jax.ShapeDtypeStruct((KK, 1), jnp.int32),
            jax.ShapeDtypeStruct((KK, 1), jnp.float32),
        ],
        interpret=interpret,
    )(rank_row, scores_row)


def _sc_gather(g, h, idx):
    n = g.shape[0]
    d = h.shape[1]
    info = plsc.get_sparse_core_info()
    nc, ns = info.num_cores, info.num_subcores
    nw = nc * ns
    rows_w = KK // nw
    ch = 16
    nch = rows_w // ch
    mesh = plsc.VectorSubcoreMesh(core_axis_name="c", subcore_axis_name="s")

    def body(g_hbm, h_hbm, idx_hbm, outg, outh, idx_v, gbuf, hbuf, semg, semh):
        wid = lax.axis_index("s") * nc + lax.axis_index("c")
        base = wid * rows_w

        def step(c, carry):
            off = base + c * ch
            pltpu.sync_copy(idx_hbm.at[pl.ds(off, ch)], idx_v)
            cpg = pltpu.async_copy(g_hbm.at[idx_v], gbuf, semg)
            cph = pltpu.async_copy(h_hbm.at[idx_v], hbuf, semh)
            cpg.wait()
            cph.wait()
            pltpu.sync_copy(gbuf, outg.at[pl.ds(off, ch)])
            pltpu.sync_copy(hbuf, outh.at[pl.ds(off, ch)])
            return carry

        lax.fori_loop(0, nch, step, 0, unroll=True)

    f = pl.kernel(
        body,
        out_type=(
            jax.ShapeDtypeStruct((KK, n), jnp.float32),
            jax.ShapeDtypeStruct((KK, d), jnp.float32),
        ),
        mesh=mesh,
        scratch_types=[
            pltpu.VMEM((ch,), jnp.int32),
            pltpu.VMEM((ch, n), jnp.float32),
            pltpu.VMEM((ch, d), jnp.float32),
            pltpu.SemaphoreType.DMA,
            pltpu.SemaphoreType.DMA,
        ],
    )
    return f(g, h, idx)


def _twohop_body(r_ref, g_ref, out_ref, acc_ref):
    k = pl.program_id(1)

    @pl.when(k == 0)
    def _():
        acc_ref[...] = jnp.zeros_like(acc_ref)

    lhs = (r_ref[...] != 0).astype(jnp.bfloat16)
    rhs = (g_ref[...] != 0).astype(jnp.bfloat16)
    acc_ref[...] += jnp.dot(lhs, rhs, preferred_element_type=jnp.float32)

    @pl.when(k == pl.num_programs(1) - 1)
    def _():
        out_ref[...] = (acc_ref[...] != 0).astype(jnp.bfloat16)


def _twohop(rgat, g, nblk=2048, kblk=512, interpret=False):
    n = g.shape[0]
    return pl.pallas_call(
        _twohop_body,
        grid=(n // nblk, n // kblk),
        in_specs=[
            pl.BlockSpec((KK, kblk), lambda j, k: (0, k)),
            pl.BlockSpec((kblk, nblk), lambda j, k: (k, j)),
        ],
        out_specs=pl.BlockSpec((KK, nblk), lambda j, k: (0, j)),
        out_shape=jax.ShapeDtypeStruct((KK, n), jnp.bfloat16),
        scratch_shapes=[pltpu.VMEM((KK, nblk), jnp.float32)],
        compiler_params=pltpu.CompilerParams(
            dimension_semantics=("parallel", "arbitrary"),
        ),
        interpret=interpret,
    )(rgat, g)


def _colsel_body(c_ref, rank_ref, out_ref, deg_ref, acc_ref):
    jb = pl.program_id(0)
    k = pl.program_id(1)
    kblk = rank_ref.shape[0]
    jblk = out_ref.shape[1]

    @pl.when(k == 0)
    def _():
        acc_ref[...] = jnp.zeros_like(acc_ref)

    lhs = c_ref[...]
    rk = rank_ref[...]
    jj = lax.broadcasted_iota(jnp.int32, (kblk, jblk), 1) + jb * jblk
    rhs = (rk == jj).astype(jnp.bfloat16)
    acc_ref[...] += jnp.dot(lhs, rhs, preferred_element_type=jnp.float32)

    @pl.when(k == pl.num_programs(1) - 1)
    def _():
        cs = acc_ref[...]
        out_ref[...] = cs
        rs = jnp.sum(cs, axis=1, keepdims=True)

        @pl.when(jb == 0)
        def _():
            deg_ref[...] = rs

        @pl.when(jb > 0)
        def _():
            deg_ref[...] += rs


def _colsel(c01, rank_col, jblk=1024, kblk=2048, interpret=False):
    n = c01.shape[1]
    return pl.pallas_call(
        _colsel_body,
        grid=(KK // jblk, n // kblk),
        in_specs=[
            pl.BlockSpec((KK, kblk), lambda j, k: (0, k)),
            pl.BlockSpec((kblk, 1), lambda j, k: (k, 0)),
        ],
        out_specs=[
            pl.BlockSpec((KK, jblk), lambda j, k: (0, j)),
            pl.BlockSpec((KK, 1), lambda j, k: (0, 0)),
        ],
        out_shape=[
            jax.ShapeDtypeStruct((KK, KK), jnp.float32),
            jax.ShapeDtypeStruct((KK, 1), jnp.float32),
        ],
        scratch_shapes=[pltpu.VMEM((KK, jblk), jnp.float32)],
        compiler_params=pltpu.CompilerParams(
            dimension_semantics=("arbitrary", "arbitrary"),
        ),
        interpret=interpret,
    )(c01, rank_col)


def _final_body(c_ref, degc_ref, degr_ref, h_ref, val_ref, gout_ref, nh_ref):
    blk, kk = c_ref.shape
    i0 = pl.program_id(0) * blk
    dinv_i = lax.rsqrt(degc_ref[...])
    dinv_j = lax.rsqrt(degr_ref[...])
    gn = c_ref[...] * dinv_i * dinv_j
    ii = lax.broadcasted_iota(jnp.int32, (blk, kk), 0) + i0
    jj = lax.broadcasted_iota(jnp.int32, (blk, kk), 1)
    eye = (ii == jj).astype(jnp.float32)
    gout_ref[...] = eye + gn
    nh_ref[...] = h_ref[...] * val_ref[...]


def _final(csub, deg_col, deg_row, hgat, val2, blk=512, interpret=False):
    d = hgat.shape[1]
    return pl.pallas_call(
        _final_body,
        grid=(KK // blk,),
        in_specs=[
            pl.BlockSpec((blk, KK), lambda i: (i, 0)),
            pl.BlockSpec((blk, 1), lambda i: (i, 0)),
            pl.BlockSpec((1, KK), lambda i: (0, 0)),
            pl.BlockSpec((blk, d), lambda i: (i, 0)),
            pl.BlockSpec((blk, 1), lambda i: (i, 0)),
        ],
        out_specs=[
            pl.BlockSpec((blk, KK), lambda i: (i, 0)),
            pl.BlockSpec((blk, d), lambda i: (i, 0)),
        ],
        out_shape=[
            jax.ShapeDtypeStruct((KK, KK), jnp.float32),
            jax.ShapeDtypeStruct((KK, d), jnp.float32),
        ],
        interpret=interpret,
    )(csub, deg_col, deg_row, hgat, val2)


def kernel(g, h, proj_w, k):
    n, d = h.shape
    w_row = proj_w.reshape(1, d)
    scores = _scores(h, w_row)
    scores_row = scores.reshape(1, n)
    rank = _rank(scores, scores_row)
    rank_row = rank.reshape(1, n)
    idx2, val2 = _select(rank_row, scores_row)
    idx = idx2.reshape(KK)
    rgat, hgat = _sc_gather(g, h, idx)
    c01 = _twohop(rgat, g)
    csub, deg = _colsel(c01, rank)
    deg_row = deg.reshape(1, KK)
    g_out, new_h = _final(csub, deg, deg_row, hgat, val2)
    return (g_out, new_h, idx)


# speedup vs baseline: 1.0019x; 1.0019x over previous
import jax
import jax.numpy as jnp
from jax import lax
from jax.experimental import pallas as pl
from jax.experimental.pallas import tpu as pltpu
from jax.experimental.pallas import tpu_sc as plsc

KK = 2048


def _scores_body(h_ref, w_ref, out_ref):
    hb = h_ref[...].astype(jnp.bfloat16).astype(jnp.float32)
    wb = w_ref[...].astype(jnp.bfloat16).astype(jnp.float32)
    logits = jnp.sum(hb * wb, axis=1, keepdims=True)
    out_ref[...] = 1.0 / (1.0 + jnp.exp(-logits))


def _scores(h, w_row, interpret=False):
    n = h.shape[0]
    return pl.pallas_call(
        _scores_body,
        out_shape=jax.ShapeDtypeStruct((n, 1), jnp.float32),
        interpret=interpret,
    )(h, w_row)


def _rank_body(sc_ref, sr_ref, out_ref):
    blk, n = sc_ref.shape[0], sr_ref.shape[1]
    i0 = pl.program_id(0) * blk
    s_i = sc_ref[...]
    s_j = sr_ref[...]
    row_i = lax.broadcasted_iota(jnp.int32, (blk, n), 0) + i0
    col_j = lax.broadcasted_iota(jnp.int32, (blk, n), 1)
    gt = (s_j > s_i).astype(jnp.int32)
    tie = jnp.logical_and(s_j == s_i, col_j < row_i).astype(jnp.int32)
    out_ref[...] = jnp.sum(gt + tie, axis=1, keepdims=True)


def _rank(scores_col, scores_row, blk=512, interpret=False):
    n = scores_col.shape[0]
    return pl.pallas_call(
        _rank_body,
        grid=(n // blk,),
        in_specs=[
            pl.BlockSpec((blk, 1), lambda i: (i, 0)),
            pl.BlockSpec((1, n), lambda i: (0, 0)),
        ],
        out_specs=pl.BlockSpec((blk, 1), lambda i: (i, 0)),
        out_shape=jax.ShapeDtypeStruct((n, 1), jnp.int32),
        interpret=interpret,
    )(scores_col, scores_row)


def _select_body(rank_ref, score_ref, idx_ref, val_ref):
    blk, n = idx_ref.shape[0], rank_ref.shape[1]
    r0 = pl.program_id(0) * blk
    rk = rank_ref[...]
    sc = score_ref[...]
    r = lax.broadcasted_iota(jnp.int32, (blk, n), 0) + r0
    j = lax.broadcasted_iota(jnp.int32, (blk, n), 1)
    eq = rk == r
    idx_ref[...] = jnp.sum(jnp.where(eq, j, 0), axis=1, keepdims=True)
    val_ref[...] = jnp.sum(jnp.where(eq, sc, 0.0), axis=1, keepdims=True)


def _select(rank_row, scores_row, blk=512, interpret=False):
    n = rank_row.shape[1]
    return pl.pallas_call(
        _select_body,
        grid=(KK // blk,),
        in_specs=[
            pl.BlockSpec((1, n), lambda i: (0, 0)),
            pl.BlockSpec((1, n), lambda i: (0, 0)),
        ],
        out_specs=[
            pl.BlockSpec((blk, 1), lambda i: (i, 0)),
            pl.BlockSpec((blk, 1), lambda i: (i, 0)),
        ],
        out_shape=[
            jax.ShapeDtypeStruct((KK, 1), jnp.int32),
            jax.ShapeDtypeStruct((KK, 1), jnp.float32),
        ],
        interpret=interpret,
    )(rank_row, scores_row)


def _sc_gather(g, h, idx):
    n = g.shape[0]
    d = h.shape[1]
    info = plsc.get_sparse_core_info()
    nc, ns = info.num_cores, info.num_subcores
    nw = nc * ns
    rows_w = KK // nw
    ch = 16
    nch = rows_w // ch
    mesh = plsc.VectorSubcoreMesh(core_axis_name="c", subcore_axis_name="s")

    def body(g_hbm, h_hbm, idx_hbm, outg, outh, idx_v, gbuf, hbuf, semg, semh):
        wid = lax.axis_index("s") * nc + lax.axis_index("c")
        base = wid * rows_w

        def step(c, carry):
            off = base + c * ch
            pltpu.sync_copy(idx_hbm.at[pl.ds(off, ch)], idx_v)
            cpg = pltpu.async_copy(g_hbm.at[idx_v], gbuf, semg)
            cph = pltpu.async_copy(h_hbm.at[idx_v], hbuf, semh)
            cpg.wait()
            cph.wait()
            pltpu.sync_copy(gbuf, outg.at[pl.ds(off, ch)])
            pltpu.sync_copy(hbuf, outh.at[pl.ds(off, ch)])
            return carry

        lax.fori_loop(0, nch, step, 0, unroll=True)

    f = pl.kernel(
        body,
        out_type=(
            jax.ShapeDtypeStruct((KK, n), jnp.float32),
            jax.ShapeDtypeStruct((KK, d), jnp.float32),
        ),
        mesh=mesh,
        scratch_types=[
            pltpu.VMEM((ch,), jnp.int32),
            pltpu.VMEM((ch, n), jnp.float32),
            pltpu.VMEM((ch, d), jnp.float32),
            pltpu.SemaphoreType.DMA,
            pltpu.SemaphoreType.DMA,
        ],
    )
    return f(g, h, idx)


def _twohop_body(r_ref, g_ref, out_ref, acc_ref):
    k = pl.program_id(1)

    @pl.when(k == 0)
    def _():
        acc_ref[...] = jnp.zeros_like(acc_ref)

    lhs = (r_ref[...] != 0).astype(jnp.bfloat16)
    rhs = (g_ref[...] != 0).astype(jnp.bfloat16)
    acc_ref[...] += jnp.dot(lhs, rhs, preferred_element_type=jnp.float32)

    @pl.when(k == pl.num_programs(1) - 1)
    def _():
        out_ref[...] = (acc_ref[...] != 0).astype(jnp.bfloat16)


def _twohop(rgat, g, nblk=2048, kblk=512, interpret=False):
    n = g.shape[0]
    return pl.pallas_call(
        _twohop_body,
        grid=(n // nblk, n // kblk),
        in_specs=[
            pl.BlockSpec((KK, kblk), lambda j, k: (0, k)),
            pl.BlockSpec((kblk, nblk), lambda j, k: (k, j)),
        ],
        out_specs=pl.BlockSpec((KK, nblk), lambda j, k: (0, j)),
        out_shape=jax.ShapeDtypeStruct((KK, n), jnp.bfloat16),
        scratch_shapes=[pltpu.VMEM((KK, nblk), jnp.float32)],
        compiler_params=pltpu.CompilerParams(
            dimension_semantics=("parallel", "arbitrary"),
        ),
        interpret=interpret,
    )(rgat, g)


def _colsel_body(c_ref, rank_ref, out_ref, deg_ref, acc_ref):
    jb = pl.program_id(0)
    k = pl.program_id(1)
    kblk = rank_ref.shape[0]
    jblk = out_ref.shape[1]

    @pl.when(k == 0)
    def _():
        acc_ref[...] = jnp.zeros_like(acc_ref)

    lhs = c_ref[...]
    rk = rank_ref[...]
    jj = lax.broadcasted_iota(jnp.int32, (kblk, jblk), 1) + jb * jblk
    rhs = (rk == jj).astype(jnp.bfloat16)
    acc_ref[...] += jnp.dot(lhs, rhs, preferred_element_type=jnp.float32)

    @pl.when(k == pl.num_programs(1) - 1)
    def _():
        cs = acc_ref[...]
        out_ref[...] = cs
        rs = jnp.sum(cs, axis=1, keepdims=True)

        @pl.when(jb == 0)
        def _():
            deg_ref[...] = rs

        @pl.when(jb > 0)
        def _():
            deg_ref[...] += rs


def _colsel(c01, rank_col, jblk=1024, kblk=1024, interpret=False):
    n = c01.shape[1]
    return pl.pallas_call(
        _colsel_body,
        grid=(KK // jblk, n // kblk),
        in_specs=[
            pl.BlockSpec((KK, kblk), lambda j, k: (0, k)),
            pl.BlockSpec((kblk, 1), lambda j, k: (k, 0)),
        ],
        out_specs=[
            pl.BlockSpec((KK, jblk), lambda j, k: (0, j)),
            pl.BlockSpec((KK, 1), lambda j, k: (0, 0)),
        ],
        out_shape=[
            jax.ShapeDtypeStruct((KK, KK), jnp.float32),
            jax.ShapeDtypeStruct((KK, 1), jnp.float32),
        ],
        scratch_shapes=[pltpu.VMEM((KK, jblk), jnp.float32)],
        compiler_params=pltpu.CompilerParams(
            dimension_semantics=("arbitrary", "arbitrary"),
        ),
        interpret=interpret,
    )(c01, rank_col)


def _final_body(c_ref, degc_ref, degr_ref, h_ref, val_ref, gout_ref, nh_ref):
    blk, kk = c_ref.shape
    i0 = pl.program_id(0) * blk
    dinv_i = lax.rsqrt(degc_ref[...])
    dinv_j = lax.rsqrt(degr_ref[...])
    gn = c_ref[...] * dinv_i * dinv_j
    ii = lax.broadcasted_iota(jnp.int32, (blk, kk), 0) + i0
    jj = lax.broadcasted_iota(jnp.int32, (blk, kk), 1)
    eye = (ii == jj).astype(jnp.float32)
    gout_ref[...] = eye + gn
    nh_ref[...] = h_ref[...] * val_ref[...]


def _final(csub, deg_col, deg_row, hgat, val2, blk=512, interpret=False):
    d = hgat.shape[1]
    return pl.pallas_call(
        _final_body,
        grid=(KK // blk,),
        in_specs=[
            pl.BlockSpec((blk, KK), lambda i: (i, 0)),
            pl.BlockSpec((blk, 1), lambda i: (i, 0)),
            pl.BlockSpec((1, KK), lambda i: (0, 0)),
            pl.BlockSpec((blk, d), lambda i: (i, 0)),
            pl.BlockSpec((blk, 1), lambda i: (i, 0)),
        ],
        out_specs=[
            pl.BlockSpec((blk, KK), lambda i: (i, 0)),
            pl.BlockSpec((blk, d), lambda i: (i, 0)),
        ],
        out_shape=[
            jax.ShapeDtypeStruct((KK, KK), jnp.float32),
            jax.ShapeDtypeStruct((KK, d), jnp.float32),
        ],
        interpret=interpret,
    )(csub, deg_col, deg_row, hgat, val2)


def kernel(g, h, proj_w, k):
    n, d = h.shape
    w_row = proj_w.reshape(1, d)
    scores = _scores(h, w_row)
    scores_row = scores.reshape(1, n)
    rank = _rank(scores, scores_row)
    rank_row = rank.reshape(1, n)
    idx2, val2 = _select(rank_row, scores_row)
    idx = idx2.reshape(KK)
    rgat, hgat = _sc_gather(g, h, idx)
    c01 = _twohop(rgat, g)
    csub, deg = _colsel(c01, rank)
    deg_row = deg.reshape(1, KK)
    g_out, new_h = _final(csub, deg, deg_row, hgat, val2)
    return (g_out, new_h, idx)
